# baseline (speedup 1.0000x reference)
import functools

import jax
import jax.numpy as jnp
from jax import lax
from jax.experimental import pallas as pl
from jax.experimental.pallas import tpu as pltpu
from jax.experimental.pallas import tpu_sc as plsc

_NC = 2
_NS = 16
_CH = 128
_IB = 16
_F = 128


@functools.lru_cache(maxsize=None)
def _make_agg(U, nch):
    mesh = plsc.VectorSubcoreMesh(core_axis_name="c", subcore_axis_name="s")
    rpt = (U // _NS + 7) // 8 * 8
    UA = _NS * rpt
    assert nch % _IB == 0
    nblk = nch // _IB
    ib2 = _IB // 2

    out_type = jax.ShapeDtypeStruct((2 * UA, _F), jnp.float32)
    scratch = (
        pltpu.VMEM((_IB, _CH), jnp.int32),
        pltpu.VMEM((_IB, _CH), jnp.int32),
        pltpu.VMEM((_CH, _F), jnp.float32),
        pltpu.VMEM((_CH, _F), jnp.float32),
        pltpu.VMEM_SHARED((UA, _F), jnp.float32),
        pltpu.SemaphoreType.DMA,
        pltpu.SemaphoreType.DMA,
        pltpu.SemaphoreType.DMA,
        pltpu.SemaphoreType.DMA,
    )

    def body(ufeat, sfeat, eu_src, es_src, eu_dst, es_dst, zf, out,
             idx_s, idx_d, rows_a, rows_b, acc, sem_ga, sem_gb, sem_sa,
             sem_sb):
        c = lax.axis_index("c")
        t = lax.axis_index("s")
        base = t * rpt

        def run_side(src_feat, src_idx_h, dst_idx_h, out_base):
            pltpu.sync_copy(zf.at[pl.ds(base, rpt)], acc.at[pl.ds(base, rpt)])
            plsc.subcore_barrier()

            def block(bk, _):
                pltpu.sync_copy(src_idx_h.at[t, pl.ds(bk * _IB, _IB)], idx_s)
                pltpu.sync_copy(dst_idx_h.at[t, pl.ds(bk * _IB, _IB)], idx_d)
                pltpu.async_copy(src_feat.at[idx_s.at[0]], rows_a, sem_ga)
                pltpu.async_copy(src_feat.at[idx_s.at[1]], rows_b, sem_gb)

                def pair(jj, _):
                    j0 = 2 * jj
                    pltpu.make_async_copy(src_feat.at[idx_s.at[j0]], rows_a,
                                          sem_ga).wait()
                    pltpu.async_copy(rows_a, acc.at[idx_d.at[j0]], sem_sa,
                                     add=True)
                    pltpu.make_async_copy(src_feat.at[idx_s.at[j0 + 1]],
                                          rows_b, sem_gb).wait()
                    pltpu.async_copy(rows_b, acc.at[idx_d.at[j0 + 1]], sem_sb,
                                     add=True)
                    pltpu.make_async_copy(rows_a, acc.at[idx_d.at[j0]],
                                          sem_sa).wait()

                    @pl.when(jj + 1 < ib2)
                    def _():
                        pltpu.async_copy(src_feat.at[idx_s.at[j0 + 2]],
                                         rows_a, sem_ga)

                    pltpu.make_async_copy(rows_b, acc.at[idx_d.at[j0 + 1]],
                                          sem_sb).wait()

                    @pl.when(jj + 1 < ib2)
                    def _():
                        pltpu.async_copy(src_feat.at[idx_s.at[j0 + 3]],
                                         rows_b, sem_gb)
                    return 0

                lax.fori_loop(0, ib2, pair, 0)
                return 0

            lax.fori_loop(0, nblk, block, 0)
            plsc.subcore_barrier()
            pltpu.sync_copy(acc.at[pl.ds(base, rpt)],
                            out.at[pl.ds(out_base + base, rpt)])

        @pl.when(c == 0)
        def _():
            run_side(sfeat, es_src, eu_dst, 0)

        @pl.when(c == 1)
        def _():
            run_side(ufeat, eu_src, es_dst, UA)

    return pl.kernel(body, out_type=out_type, mesh=mesh,
                     scratch_types=scratch)


@functools.lru_cache(maxsize=None)
def _make_counts(U, nch):
    mesh = plsc.VectorSubcoreMesh(core_axis_name="c", subcore_axis_name="s")
    rpt = (U // _NS + 7) // 8 * 8
    UA = _NS * rpt
    assert nch % _IB == 0
    nblk = nch // _IB

    out_type = jax.ShapeDtypeStruct((2 * UA, _F), jnp.float32)
    scratch = (
        pltpu.VMEM((_IB, _CH), jnp.int32),
        pltpu.VMEM((_CH, _F), jnp.float32),
        pltpu.VMEM_SHARED((UA, _F), jnp.float32),
        pltpu.SemaphoreType.DMA,
    )

    def body(eu_dst, es_dst, zf, ones_h, out, idx_d, ones_v, cnt, sem_s):
        c = lax.axis_index("c")
        t = lax.axis_index("s")
        base = t * rpt

        def run_side(dst_idx_h, out_base):
            pltpu.sync_copy(zf.at[pl.ds(base, rpt)], cnt.at[pl.ds(base, rpt)])
            pltpu.sync_copy(ones_h, ones_v)
            plsc.subcore_barrier()

            def block(bk, _):
                pltpu.sync_copy(dst_idx_h.at[t, pl.ds(bk * _IB, _IB)], idx_d)

                def fire(j, _):
                    pltpu.async_copy(ones_v, cnt.at[idx_d.at[j]], sem_s,
                                     add=True)
                    return 0

                lax.fori_loop(0, _IB, fire, 0)

                def drain(j, _):
                    pltpu.make_async_copy(ones_v, cnt.at[idx_d.at[j]],
                                          sem_s).wait()
                    return 0

                lax.fori_loop(0, _IB, drain, 0)
                return 0

            lax.fori_loop(0, nblk, block, 0)
            plsc.subcore_barrier()
            pltpu.sync_copy(cnt.at[pl.ds(base, rpt)],
                            out.at[pl.ds(out_base + base, rpt)])

        @pl.when(c == 0)
        def _():
            run_side(eu_dst, 0)

        @pl.when(c == 1)
        def _():
            run_side(es_dst, UA)

    return pl.kernel(body, out_type=out_type, mesh=mesh,
                     scratch_types=scratch)


_BLK = 2000


def _in_body(x_ref, w_ref, b_ref, o_ref):
    o_ref[0] = jnp.maximum(
        jnp.dot(x_ref[0], w_ref[0], preferred_element_type=jnp.float32)
        + b_ref[0], 0.0)


def _sage_body(h_ref, a_ref, c_ref, ws_ref, wn_ref, b_ref, o_ref):
    recip = 1.0 / jnp.maximum(c_ref[0][:, :1], 1.0)
    acc = jnp.dot(h_ref[0], ws_ref[0], preferred_element_type=jnp.float32)
    acc = acc + jnp.dot(a_ref[0] * recip, wn_ref[0],
                        preferred_element_type=jnp.float32)
    o_ref[0] = jnp.maximum(acc + b_ref[0], 0.0)


def _final_body(h_ref, a_ref, c_ref, ws_ref, wn_ref, b_ref, wo_ref, bo_ref,
                o_ref):
    recip = 1.0 / jnp.maximum(c_ref[0][:, :1], 1.0)
    acc = jnp.dot(h_ref[0], ws_ref[0], preferred_element_type=jnp.float32)
    acc = acc + jnp.dot(a_ref[0] * recip, wn_ref[0],
                        preferred_element_type=jnp.float32)
    acc = jnp.maximum(acc + b_ref[0], 0.0)
    y = jnp.dot(acc, wo_ref[0], preferred_element_type=jnp.float32) + bo_ref[0]
    n = jnp.sqrt(jnp.sum(y * y, axis=1, keepdims=True))
    o_ref[0] = y / jnp.maximum(n, 1e-12)


def _row_spec(blk, d):
    return pl.BlockSpec((1, blk, d), lambda i, j: (i, j, 0))


def _bcast_spec(shape):
    return pl.BlockSpec((1,) + shape[1:], lambda i, j: (i, 0, 0))


def _linrelu(x2, W2, b2):
    n = x2.shape[1]
    return pl.pallas_call(
        _in_body,
        grid=(2, n // _BLK),
        in_specs=[_row_spec(_BLK, x2.shape[2]), _bcast_spec(W2.shape),
                  _bcast_spec(b2.shape)],
        out_specs=_row_spec(_BLK, W2.shape[2]),
        out_shape=jax.ShapeDtypeStruct((2, n, W2.shape[2]), jnp.float32),
    )(x2, W2, b2)


def _sage(h2, a2, c2, Ws, Wn, b2):
    n = h2.shape[1]
    return pl.pallas_call(
        _sage_body,
        grid=(2, n // _BLK),
        in_specs=[_row_spec(_BLK, _F), _row_spec(_BLK, _F),
                  _row_spec(_BLK, _F), _bcast_spec(Ws.shape),
                  _bcast_spec(Wn.shape), _bcast_spec(b2.shape)],
        out_specs=_row_spec(_BLK, _F),
        out_shape=jax.ShapeDtypeStruct((2, n, _F), jnp.float32),
    )(h2, a2, c2, Ws, Wn, b2)


def _final(h2, a2, c2, Ws, Wn, b2, Wo, bo):
    n = h2.shape[1]
    dout = Wo.shape[2]
    return pl.pallas_call(
        _final_body,
        grid=(2, n // _BLK),
        in_specs=[_row_spec(_BLK, _F), _row_spec(_BLK, _F),
                  _row_spec(_BLK, _F), _bcast_spec(Ws.shape),
                  _bcast_spec(Wn.shape), _bcast_spec(b2.shape),
                  _bcast_spec(Wo.shape), _bcast_spec(bo.shape)],
        out_specs=_row_spec(_BLK, dout),
        out_shape=jax.ShapeDtypeStruct((2, n, dout), jnp.float32),
    )(h2, a2, c2, Ws, Wn, b2, Wo, bo)


def kernel(user_x, session_x, edge_user_idx, edge_session_idx, params):
    p = params
    U = user_x.shape[0]
    S = session_x.shape[0]
    E = edge_user_idx.shape[0]
    assert U == S and U % _NS == 0

    eu = edge_user_idx.astype(jnp.int32)
    es = edge_session_idx.astype(jnp.int32)

    nch = -(-E // (_NS * _CH))
    nch = -(-nch // _IB) * _IB
    pad = _NS * nch * _CH - E

    def chunked(x, val):
        return jnp.concatenate(
            [x, jnp.full((pad,), val, jnp.int32)]).reshape(_NS, nch, _CH)

    eu_src = chunked(eu, 0)
    eu_dst = chunked(eu, U)
    es_src = chunked(es, 0)
    es_dst = chunked(es, S)
    UA = _NS * ((U // _NS + 7) // 8 * 8)
    zf = jnp.zeros((UA, _F), jnp.float32)
    ones_h = jnp.ones((_CH, _F), jnp.float32)

    def stack(a, b):
        return jnp.stack([a, b])

    Win = stack(p["user_in_proj_W"], p["session_in_proj_W"])
    bin_ = stack(p["user_in_proj_b"], p["session_in_proj_b"])[:, None, :]
    h0 = _linrelu(stack(user_x, session_x), Win, bin_)

    agg1 = _make_agg(U, nch)(
        h0[0], h0[1], eu_src, es_src, eu_dst, es_dst, zf)
    agg1 = agg1.reshape(2, UA, _F)[:, :U]
    cnt = _make_counts(U, nch)(eu_dst, es_dst, zf, ones_h)
    cnt = cnt.reshape(2, UA, _F)[:, :U]

    Ws1 = stack(p["user_sage_1_self_W"], p["session_sage_1_self_W"])
    Wn1 = stack(p["user_sage_1_neigh_W"], p["session_sage_1_neigh_W"])
    b1 = stack(p["user_sage_1_self_b"] + p["user_sage_1_neigh_b"],
               p["session_sage_1_self_b"] + p["session_sage_1_neigh_b"]
               )[:, None, :]
    h1 = _sage(h0, agg1, cnt, Ws1, Wn1, b1)

    agg2 = _make_agg(U, nch)(
        h1[0], h1[1], eu_src, es_src, eu_dst, es_dst, zf)
    agg2 = agg2.reshape(2, UA, _F)[:, :U]

    Ws2 = stack(p["user_sage_2_self_W"], p["session_sage_2_self_W"])
    Wn2 = stack(p["user_sage_2_neigh_W"], p["session_sage_2_neigh_W"])
    b2 = stack(p["user_sage_2_self_b"] + p["user_sage_2_neigh_b"],
               p["session_sage_2_self_b"] + p["session_sage_2_neigh_b"]
               )[:, None, :]
    Wo = stack(p["user_out_W"], p["session_out_W"])
    bo = stack(p["user_out_b"], p["session_out_b"])[:, None, :]
    emb = _final(h1, agg2, cnt, Ws2, Wn2, b2, Wo, bo)
    return (emb[0], emb[1])


# speedup vs baseline: 1.0113x; 1.0113x over previous
import functools

import jax
import jax.numpy as jnp
from jax import lax
from jax.experimental import pallas as pl
from jax.experimental.pallas import tpu as pltpu
from jax.experimental.pallas import tpu_sc as plsc

_NC = 2
_NS = 16
_CH = 64
_IB = 32
_RB = 4
_F = 128


@functools.lru_cache(maxsize=None)
def _make_agg(U, nch):
    mesh = plsc.VectorSubcoreMesh(core_axis_name="c", subcore_axis_name="s")
    rpt = (U // _NS + 7) // 8 * 8
    UA = _NS * rpt
    assert nch % _IB == 0 and _IB % _RB == 0
    nblk = nch // _IB

    out_type = jax.ShapeDtypeStruct((2 * UA, _F), jnp.float32)
    scratch = (
        pltpu.VMEM((_IB, _CH), jnp.int32),
        pltpu.VMEM((_IB, _CH), jnp.int32),
        tuple(pltpu.VMEM((_CH, _F), jnp.float32) for _ in range(_RB)),
        pltpu.VMEM_SHARED((UA, _F), jnp.float32),
        tuple(pltpu.SemaphoreType.DMA for _ in range(_RB)),
        tuple(pltpu.SemaphoreType.DMA for _ in range(_RB)),
    )

    def body(ufeat, sfeat, eu_src, es_src, eu_dst, es_dst, zf, out,
             idx_s, idx_d, rows, acc, sg, ss):
        c = lax.axis_index("c")
        t = lax.axis_index("s")
        base = t * rpt

        def run_side(src_feat, src_idx_h, dst_idx_h, out_base):
            pltpu.sync_copy(zf.at[pl.ds(base, rpt)], acc.at[pl.ds(base, rpt)])
            plsc.subcore_barrier()

            def block(bk, _):
                pltpu.sync_copy(src_idx_h.at[t, pl.ds(bk * _IB, _IB)], idx_s)
                pltpu.sync_copy(dst_idx_h.at[t, pl.ds(bk * _IB, _IB)], idx_d)
                pltpu.async_copy(src_feat.at[idx_s.at[0]], rows[0], sg[0])
                pltpu.async_copy(src_feat.at[idx_s.at[1]], rows[1], sg[1])

                def group(jj, _):
                    j0 = jj * _RB
                    for k in range(_RB):
                        j = j0 + k
                        b = k
                        bn = (k + 2) % _RB
                        pltpu.make_async_copy(src_feat.at[idx_s.at[j]],
                                              rows[b], sg[b]).wait()
                        pltpu.async_copy(rows[b], acc.at[idx_d.at[j]],
                                         ss[b], add=True)

                        @pl.when(j + 2 < _IB)
                        def _():
                            @pl.when(j >= 2)
                            def _():
                                pltpu.make_async_copy(
                                    rows[bn], acc.at[idx_d.at[j - 2]],
                                    ss[bn]).wait()

                            pltpu.async_copy(src_feat.at[idx_s.at[j + 2]],
                                             rows[bn], sg[bn])
                    return 0

                lax.fori_loop(0, _IB // _RB, group, 0)
                for b in range(_RB):
                    pltpu.make_async_copy(rows[b],
                                          acc.at[idx_d.at[_IB - _RB + b]],
                                          ss[b]).wait()
                return 0

            lax.fori_loop(0, nblk, block, 0)
            plsc.subcore_barrier()
            pltpu.sync_copy(acc.at[pl.ds(base, rpt)],
                            out.at[pl.ds(out_base + base, rpt)])

        @pl.when(c == 0)
        def _():
            run_side(sfeat, es_src, eu_dst, 0)

        @pl.when(c == 1)
        def _():
            run_side(ufeat, eu_src, es_dst, UA)

    return pl.kernel(body, out_type=out_type, mesh=mesh,
                     scratch_types=scratch)


@functools.lru_cache(maxsize=None)
def _make_counts(U, nch):
    mesh = plsc.VectorSubcoreMesh(core_axis_name="c", subcore_axis_name="s")
    rpt = (U // _NS + 7) // 8 * 8
    UA = _NS * rpt
    assert nch % _IB == 0
    nblk = nch // _IB

    out_type = jax.ShapeDtypeStruct((2 * UA, _F), jnp.float32)
    scratch = (
        pltpu.VMEM((_IB, _CH), jnp.int32),
        pltpu.VMEM((_CH, _F), jnp.float32),
        pltpu.VMEM_SHARED((UA, _F), jnp.float32),
        pltpu.SemaphoreType.DMA,
    )

    def body(eu_dst, es_dst, zf, ones_h, out, idx_d, ones_v, cnt, sem_s):
        c = lax.axis_index("c")
        t = lax.axis_index("s")
        base = t * rpt

        def run_side(dst_idx_h, out_base):
            pltpu.sync_copy(zf.at[pl.ds(base, rpt)], cnt.at[pl.ds(base, rpt)])
            pltpu.sync_copy(ones_h, ones_v)
            plsc.subcore_barrier()

            def block(bk, _):
                pltpu.sync_copy(dst_idx_h.at[t, pl.ds(bk * _IB, _IB)], idx_d)

                def fire(j, _):
                    pltpu.async_copy(ones_v, cnt.at[idx_d.at[j]], sem_s,
                                     add=True)
                    return 0

                lax.fori_loop(0, _IB, fire, 0)

                def drain(j, _):
                    pltpu.make_async_copy(ones_v, cnt.at[idx_d.at[j]],
                                          sem_s).wait()
                    return 0

                lax.fori_loop(0, _IB, drain, 0)
                return 0

            lax.fori_loop(0, nblk, block, 0)
            plsc.subcore_barrier()
            pltpu.sync_copy(cnt.at[pl.ds(base, rpt)],
                            out.at[pl.ds(out_base + base, rpt)])

        @pl.when(c == 0)
        def _():
            run_side(eu_dst, 0)

        @pl.when(c == 1)
        def _():
            run_side(es_dst, UA)

    return pl.kernel(body, out_type=out_type, mesh=mesh,
                     scratch_types=scratch)


_BLK = 2000


def _in_body(x_ref, w_ref, b_ref, o_ref):
    o_ref[0] = jnp.maximum(
        jnp.dot(x_ref[0], w_ref[0], preferred_element_type=jnp.float32)
        + b_ref[0], 0.0)


def _sage_body(h_ref, a_ref, c_ref, ws_ref, wn_ref, b_ref, o_ref):
    recip = 1.0 / jnp.maximum(c_ref[0][:, :1], 1.0)
    acc = jnp.dot(h_ref[0], ws_ref[0], preferred_element_type=jnp.float32)
    acc = acc + jnp.dot(a_ref[0] * recip, wn_ref[0],
                        preferred_element_type=jnp.float32)
    o_ref[0] = jnp.maximum(acc + b_ref[0], 0.0)


def _final_body(h_ref, a_ref, c_ref, ws_ref, wn_ref, b_ref, wo_ref, bo_ref,
                o_ref):
    recip = 1.0 / jnp.maximum(c_ref[0][:, :1], 1.0)
    acc = jnp.dot(h_ref[0], ws_ref[0], preferred_element_type=jnp.float32)
    acc = acc + jnp.dot(a_ref[0] * recip, wn_ref[0],
                        preferred_element_type=jnp.float32)
    acc = jnp.maximum(acc + b_ref[0], 0.0)
    y = jnp.dot(acc, wo_ref[0], preferred_element_type=jnp.float32) + bo_ref[0]
    n = jnp.sqrt(jnp.sum(y * y, axis=1, keepdims=True))
    o_ref[0] = y / jnp.maximum(n, 1e-12)


def _row_spec(blk, d):
    return pl.BlockSpec((1, blk, d), lambda i, j: (i, j, 0))


def _bcast_spec(shape):
    return pl.BlockSpec((1,) + shape[1:], lambda i, j: (i, 0, 0))


def _linrelu(x2, W2, b2):
    n = x2.shape[1]
    return pl.pallas_call(
        _in_body,
        grid=(2, n // _BLK),
        in_specs=[_row_spec(_BLK, x2.shape[2]), _bcast_spec(W2.shape),
                  _bcast_spec(b2.shape)],
        out_specs=_row_spec(_BLK, W2.shape[2]),
        out_shape=jax.ShapeDtypeStruct((2, n, W2.shape[2]), jnp.float32),
    )(x2, W2, b2)


def _sage(h2, a2, c2, Ws, Wn, b2):
    n = h2.shape[1]
    return pl.pallas_call(
        _sage_body,
        grid=(2, n // _BLK),
        in_specs=[_row_spec(_BLK, _F), _row_spec(_BLK, _F),
                  _row_spec(_BLK, _F), _bcast_spec(Ws.shape),
                  _bcast_spec(Wn.shape), _bcast_spec(b2.shape)],
        out_specs=_row_spec(_BLK, _F),
        out_shape=jax.ShapeDtypeStruct((2, n, _F), jnp.float32),
    )(h2, a2, c2, Ws, Wn, b2)


def _final(h2, a2, c2, Ws, Wn, b2, Wo, bo):
    n = h2.shape[1]
    dout = Wo.shape[2]
    return pl.pallas_call(
        _final_body,
        grid=(2, n // _BLK),
        in_specs=[_row_spec(_BLK, _F), _row_spec(_BLK, _F),
                  _row_spec(_BLK, _F), _bcast_spec(Ws.shape),
                  _bcast_spec(Wn.shape), _bcast_spec(b2.shape),
                  _bcast_spec(Wo.shape), _bcast_spec(bo.shape)],
        out_specs=_row_spec(_BLK, dout),
        out_shape=jax.ShapeDtypeStruct((2, n, dout), jnp.float32),
    )(h2, a2, c2, Ws, Wn, b2, Wo, bo)


def kernel(user_x, session_x, edge_user_idx, edge_session_idx, params):
    p = params
    U = user_x.shape[0]
    S = session_x.shape[0]
    E = edge_user_idx.shape[0]
    assert U == S and U % _NS == 0

    eu = edge_user_idx.astype(jnp.int32)
    es = edge_session_idx.astype(jnp.int32)

    nch = -(-E // (_NS * _CH))
    nch = -(-nch // _IB) * _IB
    pad = _NS * nch * _CH - E

    def chunked(x, val):
        return jnp.concatenate(
            [x, jnp.full((pad,), val, jnp.int32)]).reshape(_NS, nch, _CH)

    eu_src = chunked(eu, 0)
    eu_dst = chunked(eu, U)
    es_src = chunked(es, 0)
    es_dst = chunked(es, S)
    UA = _NS * ((U // _NS + 7) // 8 * 8)
    zf = jnp.zeros((UA, _F), jnp.float32)
    ones_h = jnp.ones((_CH, _F), jnp.float32)

    def stack(a, b):
        return jnp.stack([a, b])

    Win = stack(p["user_in_proj_W"], p["session_in_proj_W"])
    bin_ = stack(p["user_in_proj_b"], p["session_in_proj_b"])[:, None, :]
    h0 = _linrelu(stack(user_x, session_x), Win, bin_)

    agg1 = _make_agg(U, nch)(
        h0[0], h0[1], eu_src, es_src, eu_dst, es_dst, zf)
    agg1 = agg1.reshape(2, UA, _F)[:, :U]
    cnt = _make_counts(U, nch)(eu_dst, es_dst, zf, ones_h)
    cnt = cnt.reshape(2, UA, _F)[:, :U]

    Ws1 = stack(p["user_sage_1_self_W"], p["session_sage_1_self_W"])
    Wn1 = stack(p["user_sage_1_neigh_W"], p["session_sage_1_neigh_W"])
    b1 = stack(p["user_sage_1_self_b"] + p["user_sage_1_neigh_b"],
               p["session_sage_1_self_b"] + p["session_sage_1_neigh_b"]
               )[:, None, :]
    h1 = _sage(h0, agg1, cnt, Ws1, Wn1, b1)

    agg2 = _make_agg(U, nch)(
        h1[0], h1[1], eu_src, es_src, eu_dst, es_dst, zf)
    agg2 = agg2.reshape(2, UA, _F)[:, :U]

    Ws2 = stack(p["user_sage_2_self_W"], p["session_sage_2_self_W"])
    Wn2 = stack(p["user_sage_2_neigh_W"], p["session_sage_2_neigh_W"])
    b2 = stack(p["user_sage_2_self_b"] + p["user_sage_2_neigh_b"],
               p["session_sage_2_self_b"] + p["session_sage_2_neigh_b"]
               )[:, None, :]
    Wo = stack(p["user_out_W"], p["session_out_W"])
    bo = stack(p["user_out_b"], p["session_out_b"])[:, None, :]
    emb = _final(h1, agg2, cnt, Ws2, Wn2, b2, Wo, bo)
    return (emb[0], emb[1])


# speedup vs baseline: 1.0781x; 1.0661x over previous
import functools

import jax
import jax.numpy as jnp
from jax import lax
from jax.experimental import pallas as pl
from jax.experimental.pallas import tpu as pltpu
from jax.experimental.pallas import tpu_sc as plsc

_NC = 2
_NS = 16
_CH = 128
_IB = 16
_F = 128


@functools.lru_cache(maxsize=None)
def _make_agg(U, nch):
    mesh = plsc.VectorSubcoreMesh(core_axis_name="c", subcore_axis_name="s")
    rpt = (U // _NS + 7) // 8 * 8
    UA = _NS * rpt
    assert nch % _IB == 0
    nblk = nch // _IB
    ib2 = _IB // 2

    out_type = jax.ShapeDtypeStruct((2 * UA, _F), jnp.float32)
    scratch = (
        pltpu.VMEM((_IB, _CH), jnp.int32),
        pltpu.VMEM((_IB, _CH), jnp.int32),
        pltpu.VMEM((_CH, _F), jnp.float32),
        pltpu.VMEM((_CH, _F), jnp.float32),
        pltpu.VMEM_SHARED((UA, _F), jnp.float32),
        pltpu.SemaphoreType.DMA,
        pltpu.SemaphoreType.DMA,
    )

    def body(ufeat, sfeat, eu_src, es_src, eu_dst, es_dst, zf, out,
             idx_s, idx_d, rows_a, rows_b, acc, sem_a, sem_b):
        c = lax.axis_index("c")
        t = lax.axis_index("s")
        base = t * rpt

        def run_side(src_feat, src_idx_h, dst_idx_h, out_base):
            pltpu.sync_copy(zf.at[pl.ds(base, rpt)], acc.at[pl.ds(base, rpt)])
            plsc.subcore_barrier()

            def block(bk, _):
                pltpu.sync_copy(src_idx_h.at[t, pl.ds(bk * _IB, _IB)], idx_s)
                pltpu.sync_copy(dst_idx_h.at[t, pl.ds(bk * _IB, _IB)], idx_d)
                pltpu.async_copy(src_feat.at[idx_s.at[0]], rows_a, sem_a)

                def pair(jj, _):
                    j0 = 2 * jj
                    pltpu.async_copy(src_feat.at[idx_s.at[j0 + 1]], rows_b,
                                     sem_b)
                    pltpu.make_async_copy(src_feat.at[idx_s.at[j0]], rows_a,
                                          sem_a).wait()
                    pltpu.sync_copy(rows_a, acc.at[idx_d.at[j0]], add=True)

                    @pl.when(jj + 1 < ib2)
                    def _():
                        pltpu.async_copy(src_feat.at[idx_s.at[j0 + 2]],
                                         rows_a, sem_a)

                    pltpu.make_async_copy(src_feat.at[idx_s.at[j0 + 1]],
                                          rows_b, sem_b).wait()
                    pltpu.sync_copy(rows_b, acc.at[idx_d.at[j0 + 1]],
                                    add=True)
                    return 0

                lax.fori_loop(0, ib2, pair, 0)
                return 0

            lax.fori_loop(0, nblk, block, 0)
            plsc.subcore_barrier()
            pltpu.sync_copy(acc.at[pl.ds(base, rpt)],
                            out.at[pl.ds(out_base + base, rpt)])

        @pl.when(c == 0)
        def _():
            run_side(sfeat, es_src, eu_dst, 0)

        @pl.when(c == 1)
        def _():
            run_side(ufeat, eu_src, es_dst, UA)

    return pl.kernel(body, out_type=out_type, mesh=mesh,
                     scratch_types=scratch)


@functools.lru_cache(maxsize=None)
def _make_counts(U, nch):
    mesh = plsc.VectorSubcoreMesh(core_axis_name="c", subcore_axis_name="s")
    rpt = (U // _NS + 7) // 8 * 8
    UA = _NS * rpt
    assert nch % _IB == 0
    nblk = nch // _IB

    out_type = jax.ShapeDtypeStruct((2 * UA, _F), jnp.float32)
    scratch = (
        pltpu.VMEM((_IB, _CH), jnp.int32),
        pltpu.VMEM((_CH, _F), jnp.float32),
        pltpu.VMEM_SHARED((UA, _F), jnp.float32),
        pltpu.SemaphoreType.DMA,
    )

    def body(eu_dst, es_dst, zf, ones_h, out, idx_d, ones_v, cnt, sem_s):
        c = lax.axis_index("c")
        t = lax.axis_index("s")
        base = t * rpt

        def run_side(dst_idx_h, out_base):
            pltpu.sync_copy(zf.at[pl.ds(base, rpt)], cnt.at[pl.ds(base, rpt)])
            pltpu.sync_copy(ones_h, ones_v)
            plsc.subcore_barrier()

            def block(bk, _):
                pltpu.sync_copy(dst_idx_h.at[t, pl.ds(bk * _IB, _IB)], idx_d)

                def fire(j, _):
                    pltpu.async_copy(ones_v, cnt.at[idx_d.at[j]], sem_s,
                                     add=True)
                    return 0

                lax.fori_loop(0, _IB, fire, 0)

                def drain(j, _):
                    pltpu.make_async_copy(ones_v, cnt.at[idx_d.at[j]],
                                          sem_s).wait()
                    return 0

                lax.fori_loop(0, _IB, drain, 0)
                return 0

            lax.fori_loop(0, nblk, block, 0)
            plsc.subcore_barrier()
            pltpu.sync_copy(cnt.at[pl.ds(base, rpt)],
                            out.at[pl.ds(out_base + base, rpt)])

        @pl.when(c == 0)
        def _():
            run_side(eu_dst, 0)

        @pl.when(c == 1)
        def _():
            run_side(es_dst, UA)

    return pl.kernel(body, out_type=out_type, mesh=mesh,
                     scratch_types=scratch)


_BLK = 2000


def _in_body(x_ref, w_ref, b_ref, o_ref):
    o_ref[0] = jnp.maximum(
        jnp.dot(x_ref[0], w_ref[0], preferred_element_type=jnp.float32)
        + b_ref[0], 0.0)


def _sage_body(h_ref, a_ref, c_ref, ws_ref, wn_ref, b_ref, o_ref):
    recip = 1.0 / jnp.maximum(c_ref[0][:, :1], 1.0)
    acc = jnp.dot(h_ref[0], ws_ref[0], preferred_element_type=jnp.float32)
    acc = acc + jnp.dot(a_ref[0].astype(jnp.float32) * recip, wn_ref[0],
                        preferred_element_type=jnp.float32)
    o_ref[0] = jnp.maximum(acc + b_ref[0], 0.0)


def _final_body(h_ref, a_ref, c_ref, ws_ref, wn_ref, b_ref, wo_ref, bo_ref,
                o_ref):
    recip = 1.0 / jnp.maximum(c_ref[0][:, :1], 1.0)
    acc = jnp.dot(h_ref[0], ws_ref[0], preferred_element_type=jnp.float32)
    acc = acc + jnp.dot(a_ref[0].astype(jnp.float32) * recip, wn_ref[0],
                        preferred_element_type=jnp.float32)
    acc = jnp.maximum(acc + b_ref[0], 0.0)
    y = jnp.dot(acc, wo_ref[0], preferred_element_type=jnp.float32) + bo_ref[0]
    n = jnp.sqrt(jnp.sum(y * y, axis=1, keepdims=True))
    o_ref[0] = y / jnp.maximum(n, 1e-12)


def _row_spec(blk, d):
    return pl.BlockSpec((1, blk, d), lambda i, j: (i, j, 0))


def _bcast_spec(shape):
    return pl.BlockSpec((1,) + shape[1:], lambda i, j: (i, 0, 0))


def _linrelu(x2, W2, b2):
    n = x2.shape[1]
    return pl.pallas_call(
        _in_body,
        grid=(2, n // _BLK),
        in_specs=[_row_spec(_BLK, x2.shape[2]), _bcast_spec(W2.shape),
                  _bcast_spec(b2.shape)],
        out_specs=_row_spec(_BLK, W2.shape[2]),
        out_shape=jax.ShapeDtypeStruct((2, n, W2.shape[2]), jnp.float32),
    )(x2, W2, b2)


def _sage(h2, a2, c2, Ws, Wn, b2):
    n = h2.shape[1]
    return pl.pallas_call(
        _sage_body,
        grid=(2, n // _BLK),
        in_specs=[_row_spec(_BLK, _F), _row_spec(_BLK, _F),
                  _row_spec(_BLK, _F), _bcast_spec(Ws.shape),
                  _bcast_spec(Wn.shape), _bcast_spec(b2.shape)],
        out_specs=_row_spec(_BLK, _F),
        out_shape=jax.ShapeDtypeStruct((2, n, _F), jnp.float32),
    )(h2, a2, c2, Ws, Wn, b2)


def _final(h2, a2, c2, Ws, Wn, b2, Wo, bo):
    n = h2.shape[1]
    dout = Wo.shape[2]
    return pl.pallas_call(
        _final_body,
        grid=(2, n // _BLK),
        in_specs=[_row_spec(_BLK, _F), _row_spec(_BLK, _F),
                  _row_spec(_BLK, _F), _bcast_spec(Ws.shape),
                  _bcast_spec(Wn.shape), _bcast_spec(b2.shape),
                  _bcast_spec(Wo.shape), _bcast_spec(bo.shape)],
        out_specs=_row_spec(_BLK, dout),
        out_shape=jax.ShapeDtypeStruct((2, n, dout), jnp.float32),
    )(h2, a2, c2, Ws, Wn, b2, Wo, bo)


def kernel(user_x, session_x, edge_user_idx, edge_session_idx, params):
    p = params
    U = user_x.shape[0]
    S = session_x.shape[0]
    E = edge_user_idx.shape[0]
    assert U == S and U % _NS == 0

    eu = edge_user_idx.astype(jnp.int32)
    es = edge_session_idx.astype(jnp.int32)

    nch = -(-E // (_NS * _CH))
    nch = -(-nch // _IB) * _IB
    pad = _NS * nch * _CH - E

    def chunked(x, val):
        return jnp.concatenate(
            [x, jnp.full((pad,), val, jnp.int32)]).reshape(_NS, nch, _CH)

    eu_src = chunked(eu, 0)
    eu_dst = chunked(eu, U)
    es_src = chunked(es, 0)
    es_dst = chunked(es, S)
    UA = _NS * ((U // _NS + 7) // 8 * 8)
    zf = jnp.zeros((UA, _F), jnp.float32)
    ones_h = jnp.ones((_CH, _F), jnp.float32)

    def stack(a, b):
        return jnp.stack([a, b])

    Win = stack(p["user_in_proj_W"], p["session_in_proj_W"])
    bin_ = stack(p["user_in_proj_b"], p["session_in_proj_b"])[:, None, :]
    h0 = _linrelu(stack(user_x, session_x), Win, bin_)

    agg1 = _make_agg(U, nch)(
        h0[0], h0[1], eu_src, es_src, eu_dst, es_dst, zf)
    agg1 = agg1.reshape(2, UA, _F)[:, :U]
    cnt = _make_counts(U, nch)(eu_dst, es_dst, zf, ones_h)
    cnt = cnt.reshape(2, UA, _F)[:, :U]

    Ws1 = stack(p["user_sage_1_self_W"], p["session_sage_1_self_W"])
    Wn1 = stack(p["user_sage_1_neigh_W"], p["session_sage_1_neigh_W"])
    b1 = stack(p["user_sage_1_self_b"] + p["user_sage_1_neigh_b"],
               p["session_sage_1_self_b"] + p["session_sage_1_neigh_b"]
               )[:, None, :]
    h1 = _sage(h0, agg1, cnt, Ws1, Wn1, b1)

    agg2 = _make_agg(U, nch)(
        h1[0], h1[1], eu_src, es_src, eu_dst, es_dst, zf)
    agg2 = agg2.reshape(2, UA, _F)[:, :U]

    Ws2 = stack(p["user_sage_2_self_W"], p["session_sage_2_self_W"])
    Wn2 = stack(p["user_sage_2_neigh_W"], p["session_sage_2_neigh_W"])
    b2 = stack(p["user_sage_2_self_b"] + p["user_sage_2_neigh_b"],
               p["session_sage_2_self_b"] + p["session_sage_2_neigh_b"]
               )[:, None, :]
    Wo = stack(p["user_out_W"], p["session_out_W"])
    bo = stack(p["user_out_b"], p["session_out_b"])[:, None, :]
    emb = _final(h1, agg2, cnt, Ws2, Wn2, b2, Wo, bo)
    return (emb[0], emb[1])
